# Initial kernel scaffold; baseline (speedup 1.0000x reference)
import jax
import jax.numpy as jnp
from jax.experimental import pallas as pl


def kernel(hidden_states, W_router, W_up, W_down):
    raise NotImplementedError("write your pallas kernel here")


# speedup vs baseline: 2.0402x; 2.0402x over previous
import jax
import jax.numpy as jnp
from jax.experimental import pallas as pl
from jax.experimental.pallas import tpu as pltpu

NE = 8
TOPK = 2
H = 1024
INTER = 4096
KI_BLK = 1024
KI = INTER // KI_BLK


def _gelu_exact(x):
    a = x * 0.7071067811865476
    z = jnp.abs(a)
    t = 1.0 / (1.0 + 0.3275911 * z)
    poly = ((((1.061405429 * t - 1.453152027) * t + 1.421413741) * t
             - 0.284496736) * t + 0.254829592) * t
    y = 1.0 - poly * jnp.exp(-z * z)
    erf = jnp.sign(a) * y
    return 0.5 * x * (1.0 + erf)


def _router_kernel(x_ref, wr_ref, w_ref, aux_ref):
    n = x_ref.shape[0]
    x = x_ref[...]
    logits = jax.lax.dot_general(
        x, wr_ref[...], (((1,), (1,)), ((), ())),
        preferred_element_type=jnp.float32)
    m = jnp.max(logits, axis=-1, keepdims=True)
    ex = jnp.exp(logits - m)
    probs = ex / jnp.sum(ex, axis=-1, keepdims=True)

    lane = jax.lax.broadcasted_iota(jnp.int32, (n, NE), 1)
    m1 = jnp.max(probs, axis=-1, keepdims=True)
    a1 = jnp.min(jnp.where(probs >= m1, lane, NE), axis=-1, keepdims=True)
    oh1 = (lane == a1).astype(jnp.float32)
    p2 = jnp.where(lane == a1, -1.0, probs)
    m2 = jnp.max(p2, axis=-1, keepdims=True)
    a2 = jnp.min(jnp.where(p2 >= m2, lane, NE), axis=-1, keepdims=True)
    oh2 = (lane == a2).astype(jnp.float32)

    w_ref[...] = (m1 * oh1 + m2 * oh2) / (m1 + m2)

    counts = jnp.sum(oh1 + oh2, axis=0)
    f = counts / (n * TOPK)
    pmean = jnp.mean(probs, axis=0)
    aux_ref[...] = jnp.full((1, 1), NE * jnp.sum(f * pmean), jnp.float32)


def _experts_kernel(x_ref, w_ref, wup_ref, wdn_ref, out_ref, acc_ref):
    e = pl.program_id(0)
    ki = pl.program_id(1)

    @pl.when((e == 0) & (ki == 0))
    def _():
        acc_ref[...] = jnp.zeros_like(acc_ref)

    xb = x_ref[...].astype(jnp.bfloat16)
    wup = wup_ref[0].astype(jnp.bfloat16)
    h = jax.lax.dot_general(
        xb, wup, (((1,), (1,)), ((), ())),
        preferred_element_type=jnp.float32)
    h = _gelu_exact(h)
    wdn = wdn_ref[0].astype(jnp.bfloat16)
    part = jax.lax.dot_general(
        h.astype(jnp.bfloat16), wdn, (((1,), (1,)), ((), ())),
        preferred_element_type=jnp.float32)

    lane = jax.lax.broadcasted_iota(jnp.int32, w_ref.shape, 1)
    g = jnp.sum(w_ref[...] * (lane == e).astype(jnp.float32),
                axis=1, keepdims=True)
    acc_ref[...] += g * part

    @pl.when((e == NE - 1) & (ki == KI - 1))
    def _():
        out_ref[...] = acc_ref[...]


def kernel(hidden_states, W_router, W_up, W_down):
    batch, seq, hidden = hidden_states.shape
    n = batch * seq
    flat = hidden_states.reshape(n, hidden)

    w_gates, aux = pl.pallas_call(
        _router_kernel,
        out_shape=[
            jax.ShapeDtypeStruct((n, NE), jnp.float32),
            jax.ShapeDtypeStruct((1, 1), jnp.float32),
        ],
    )(flat, W_router)

    out = pl.pallas_call(
        _experts_kernel,
        grid=(NE, KI),
        in_specs=[
            pl.BlockSpec((n, H), lambda e, ki: (0, 0)),
            pl.BlockSpec((n, NE), lambda e, ki: (0, 0)),
            pl.BlockSpec((1, KI_BLK, H), lambda e, ki: (e, ki, 0)),
            pl.BlockSpec((1, H, KI_BLK), lambda e, ki: (e, 0, ki)),
        ],
        out_specs=pl.BlockSpec((n, H), lambda e, ki: (0, 0)),
        out_shape=jax.ShapeDtypeStruct((n, H), jnp.float32),
        scratch_shapes=[pltpu.VMEM((n, H), jnp.float32)],
    )(flat, w_gates, W_up, W_down)

    return out.reshape(batch, seq, hidden), aux[0, 0]


# speedup vs baseline: 2.9489x; 1.4454x over previous
import jax
import jax.numpy as jnp
from jax.experimental import pallas as pl
from jax.experimental.pallas import tpu as pltpu

NE = 8
TOPK = 2
H = 1024
INTER = 4096
KI_BLK = 1024
KI = INTER // KI_BLK


def _gelu(x):
    inner = 0.7978845608028654 * (x + 0.044715 * (x * x * x))
    return 0.5 * x * (1.0 + jnp.tanh(inner))


def _router_kernel(x_ref, wr_ref, w_ref, aux_ref):
    n = x_ref.shape[0]
    x = x_ref[...]
    logits = jax.lax.dot_general(
        x, wr_ref[...], (((1,), (1,)), ((), ())),
        preferred_element_type=jnp.float32)
    m = jnp.max(logits, axis=-1, keepdims=True)
    ex = jnp.exp(logits - m)
    probs = ex / jnp.sum(ex, axis=-1, keepdims=True)

    lane = jax.lax.broadcasted_iota(jnp.int32, (n, NE), 1)
    m1 = jnp.max(probs, axis=-1, keepdims=True)
    a1 = jnp.min(jnp.where(probs >= m1, lane, NE), axis=-1, keepdims=True)
    oh1 = (lane == a1).astype(jnp.float32)
    p2 = jnp.where(lane == a1, -1.0, probs)
    m2 = jnp.max(p2, axis=-1, keepdims=True)
    a2 = jnp.min(jnp.where(p2 >= m2, lane, NE), axis=-1, keepdims=True)
    oh2 = (lane == a2).astype(jnp.float32)

    w_ref[...] = (m1 * oh1 + m2 * oh2) / (m1 + m2)

    counts = jnp.sum(oh1 + oh2, axis=0)
    f = counts / (n * TOPK)
    pmean = jnp.mean(probs, axis=0)
    aux_ref[...] = jnp.full((1, 1), NE * jnp.sum(f * pmean), jnp.float32)


def _experts_kernel(x_ref, w_ref, wup_ref, wdn_ref, out_ref, acc_ref):
    e = pl.program_id(0)
    ki = pl.program_id(1)

    @pl.when((e == 0) & (ki == 0))
    def _():
        acc_ref[...] = jnp.zeros_like(acc_ref)

    xb = x_ref[...].astype(jnp.bfloat16)
    wup = wup_ref[0].astype(jnp.bfloat16)
    h = jax.lax.dot_general(
        xb, wup, (((1,), (1,)), ((), ())),
        preferred_element_type=jnp.float32)
    h = _gelu(h)
    wdn = wdn_ref[0].astype(jnp.bfloat16)
    part = jax.lax.dot_general(
        h.astype(jnp.bfloat16), wdn, (((1,), (1,)), ((), ())),
        preferred_element_type=jnp.float32)

    lane = jax.lax.broadcasted_iota(jnp.int32, w_ref.shape, 1)
    g = jnp.sum(w_ref[...] * (lane == e).astype(jnp.float32),
                axis=1, keepdims=True)
    acc_ref[...] += g * part

    @pl.when((e == NE - 1) & (ki == KI - 1))
    def _():
        out_ref[...] = acc_ref[...]


def kernel(hidden_states, W_router, W_up, W_down):
    batch, seq, hidden = hidden_states.shape
    n = batch * seq
    flat = hidden_states.reshape(n, hidden)

    w_gates, aux = pl.pallas_call(
        _router_kernel,
        out_shape=[
            jax.ShapeDtypeStruct((n, NE), jnp.float32),
            jax.ShapeDtypeStruct((1, 1), jnp.float32),
        ],
    )(flat, W_router)

    out = pl.pallas_call(
        _experts_kernel,
        grid=(NE, KI),
        in_specs=[
            pl.BlockSpec((n, H), lambda e, ki: (0, 0)),
            pl.BlockSpec((n, NE), lambda e, ki: (0, 0)),
            pl.BlockSpec((1, KI_BLK, H), lambda e, ki: (e, ki, 0)),
            pl.BlockSpec((1, H, KI_BLK), lambda e, ki: (e, 0, ki)),
        ],
        out_specs=pl.BlockSpec((n, H), lambda e, ki: (0, 0)),
        out_shape=jax.ShapeDtypeStruct((n, H), jnp.float32),
        scratch_shapes=[pltpu.VMEM((n, H), jnp.float32)],
    )(flat, w_gates, W_up, W_down)

    return out.reshape(batch, seq, hidden), aux[0, 0]


# speedup vs baseline: 3.4214x; 1.1602x over previous
import jax
import jax.numpy as jnp
from jax.experimental import pallas as pl
from jax.experimental.pallas import tpu as pltpu

NE = 8
TOPK = 2
H = 1024
INTER = 4096
KI_BLK = 1024
KI = INTER // KI_BLK
T = 256
N = 2048
R = N * TOPK + NE * T
RT = R // T
RT_PAD = 32
GR = 1024


def _gelu(x):
    inner = 0.7978845608028654 * (x + 0.044715 * (x * x * x))
    return 0.5 * x * (1.0 + jnp.tanh(inner))


def _router_kernel(x_ref, wr_ref, w1_ref, w2_ref, pos1_ref, pos2_ref,
                   te_ref, tot_ref, aux_ref):
    n = x_ref.shape[0]
    logits = jax.lax.dot_general(
        x_ref[...], wr_ref[...], (((1,), (1,)), ((), ())),
        preferred_element_type=jnp.float32)
    m = jnp.max(logits, axis=-1, keepdims=True)
    ex = jnp.exp(logits - m)
    probs = ex / jnp.sum(ex, axis=-1, keepdims=True)

    lane = jax.lax.broadcasted_iota(jnp.int32, (n, NE), 1)
    m1 = jnp.max(probs, axis=-1, keepdims=True)
    a1 = jnp.min(jnp.where(probs >= m1, lane, NE), axis=-1, keepdims=True)
    oh1 = (lane == a1).astype(jnp.float32)
    p2 = jnp.where(lane == a1, -1.0, probs)
    m2 = jnp.max(p2, axis=-1, keepdims=True)
    a2 = jnp.min(jnp.where(p2 >= m2, lane, NE), axis=-1, keepdims=True)
    oh2 = (lane == a2).astype(jnp.float32)

    s = m1 + m2
    w1_ref[...] = m1 / s
    w2_ref[...] = m2 / s

    c = oh1 + oh2
    ri = jax.lax.broadcasted_iota(jnp.int32, (n, n), 0)
    ci = jax.lax.broadcasted_iota(jnp.int32, (n, n), 1)
    tri = (ci < ri).astype(jnp.float32)
    rank = jax.lax.dot_general(
        tri, c, (((1,), (0,)), ((), ())),
        preferred_element_type=jnp.float32)
    counts = jnp.sum(c, axis=0, keepdims=True)
    pc = jnp.ceil(counts / T) * T
    er = jax.lax.broadcasted_iota(jnp.int32, (NE, NE), 0)
    ec = jax.lax.broadcasted_iota(jnp.int32, (NE, NE), 1)
    off = jax.lax.dot_general(
        pc, (er < ec).astype(jnp.float32), (((1,), (0,)), ((), ())),
        preferred_element_type=jnp.float32)
    ends = jax.lax.dot_general(
        pc, (er <= ec).astype(jnp.float32), (((1,), (0,)), ((), ())),
        preferred_element_type=jnp.float32)

    base = off + rank
    pos1_ref[...] = jnp.sum(base * oh1, axis=1, keepdims=True).astype(jnp.int32)
    pos2_ref[...] = jnp.sum(base * oh2, axis=1, keepdims=True).astype(jnp.int32)

    rt = (jax.lax.broadcasted_iota(jnp.int32, (RT_PAD, NE), 0) * T).astype(
        jnp.float32)
    te = jnp.sum((rt >= ends).astype(jnp.int32), axis=1, keepdims=True)
    te_ref[...] = jnp.minimum(te, NE - 1)
    tot_ref[...] = ends[:, NE - 1:NE].astype(jnp.int32)

    f = counts / (n * TOPK)
    pmean = jnp.mean(probs, axis=0, keepdims=True)
    aux_ref[...] = jnp.full((1, 1), NE * jnp.sum(f * pmean), jnp.float32)


def _gather_kernel(pos1_ref, pos2_ref, x_ref, xs_ref):
    j = pl.program_id(0)
    ji = jax.lax.broadcasted_iota(jnp.int32, (GR, N), 0) + j * GR
    P = ((ji == pos1_ref[...]) | (ji == pos2_ref[...])).astype(jnp.bfloat16)
    xs_ref[...] = jax.lax.dot_general(
        P, x_ref[...].astype(jnp.bfloat16), (((1,), (0,)), ((), ())),
        preferred_element_type=jnp.float32).astype(jnp.bfloat16)


def _up_kernel(te_ref, tot_ref, xs_ref, wup_ref, h_ref):
    r = pl.program_id(0)

    @pl.when(r * T < tot_ref[0])
    def _():
        wup = wup_ref[0].astype(jnp.bfloat16)
        h = jax.lax.dot_general(
            xs_ref[...], wup, (((1,), (1,)), ((), ())),
            preferred_element_type=jnp.float32)
        h_ref[...] = _gelu(h).astype(jnp.bfloat16)


def _down_kernel(te_ref, tot_ref, h_ref, wdn_ref,
                 pos1_ref, pos2_ref, w1_ref, w2_ref, y_ref):
    r = pl.program_id(0)

    @pl.when(r * T < tot_ref[0])
    def _():
        wdn = wdn_ref[0].astype(jnp.bfloat16)
        part = jax.lax.dot_general(
            h_ref[...], wdn, (((1,), (1,)), ((), ())),
            preferred_element_type=jnp.float32)
        ji = jax.lax.broadcasted_iota(jnp.int32, (T, N), 0) + r * T
        wrow = jnp.sum(
            jnp.where(ji == pos1_ref[...], w1_ref[...], 0.0)
            + jnp.where(ji == pos2_ref[...], w2_ref[...], 0.0),
            axis=1, keepdims=True)
        y_ref[...] = (part * wrow).astype(jnp.bfloat16)


def _combine_kernel(pos1_ref, pos2_ref, y_ref, out_ref):
    j = pl.program_id(0)
    ji = jax.lax.broadcasted_iota(jnp.int32, (N, GR), 1) + j * GR
    C = ((ji == pos1_ref[...]) | (ji == pos2_ref[...])).astype(jnp.bfloat16)
    part = jax.lax.dot_general(
        C, y_ref[...], (((1,), (0,)), ((), ())),
        preferred_element_type=jnp.float32)

    @pl.when(j == 0)
    def _():
        out_ref[...] = part

    @pl.when(j > 0)
    def _():
        out_ref[...] += part


def kernel(hidden_states, W_router, W_up, W_down):
    batch, seq, hidden = hidden_states.shape
    n = batch * seq
    flat = hidden_states.reshape(n, hidden)

    w1c, w2c, pos1c, pos2c, te, tot, aux = pl.pallas_call(
        _router_kernel,
        out_shape=[
            jax.ShapeDtypeStruct((n, 1), jnp.float32),
            jax.ShapeDtypeStruct((n, 1), jnp.float32),
            jax.ShapeDtypeStruct((n, 1), jnp.int32),
            jax.ShapeDtypeStruct((n, 1), jnp.int32),
            jax.ShapeDtypeStruct((RT_PAD, 1), jnp.int32),
            jax.ShapeDtypeStruct((1, 1), jnp.int32),
            jax.ShapeDtypeStruct((1, 1), jnp.float32),
        ],
    )(flat, W_router)

    pos1r = pos1c.reshape(1, n)
    pos2r = pos2c.reshape(1, n)
    w1r = w1c.reshape(1, n)
    w2r = w2c.reshape(1, n)

    xs = pl.pallas_call(
        _gather_kernel,
        grid=(R // GR,),
        in_specs=[
            pl.BlockSpec((1, n), lambda j: (0, 0)),
            pl.BlockSpec((1, n), lambda j: (0, 0)),
            pl.BlockSpec((n, H), lambda j: (0, 0)),
        ],
        out_specs=pl.BlockSpec((GR, H), lambda j: (j, 0)),
        out_shape=jax.ShapeDtypeStruct((R, H), jnp.bfloat16),
    )(pos1r, pos2r, flat)

    te_flat = te.reshape(RT_PAD)
    tot_flat = tot.reshape(1)

    h = pl.pallas_call(
        _up_kernel,
        grid_spec=pltpu.PrefetchScalarGridSpec(
            num_scalar_prefetch=2,
            grid=(RT,),
            in_specs=[
                pl.BlockSpec((T, H), lambda r, te, tot: (r, 0)),
                pl.BlockSpec((1, INTER, H), lambda r, te, tot: (te[r], 0, 0)),
            ],
            out_specs=pl.BlockSpec((T, INTER), lambda r, te, tot: (r, 0)),
        ),
        out_shape=jax.ShapeDtypeStruct((R, INTER), jnp.bfloat16),
    )(te_flat, tot_flat, xs, W_up)

    y = pl.pallas_call(
        _down_kernel,
        grid_spec=pltpu.PrefetchScalarGridSpec(
            num_scalar_prefetch=2,
            grid=(RT,),
            in_specs=[
                pl.BlockSpec((T, INTER), lambda r, te, tot: (r, 0)),
                pl.BlockSpec((1, H, INTER), lambda r, te, tot: (te[r], 0, 0)),
                pl.BlockSpec((1, n), lambda r, te, tot: (0, 0)),
                pl.BlockSpec((1, n), lambda r, te, tot: (0, 0)),
                pl.BlockSpec((1, n), lambda r, te, tot: (0, 0)),
                pl.BlockSpec((1, n), lambda r, te, tot: (0, 0)),
            ],
            out_specs=pl.BlockSpec((T, H), lambda r, te, tot: (r, 0)),
        ),
        out_shape=jax.ShapeDtypeStruct((R, H), jnp.bfloat16),
    )(te_flat, tot_flat, h, W_down, pos1r, pos2r, w1r, w2r)

    out = pl.pallas_call(
        _combine_kernel,
        grid=(R // GR,),
        in_specs=[
            pl.BlockSpec((n, 1), lambda j: (0, 0)),
            pl.BlockSpec((n, 1), lambda j: (0, 0)),
            pl.BlockSpec((GR, H), lambda j: (j, 0)),
        ],
        out_specs=pl.BlockSpec((n, H), lambda j: (0, 0)),
        out_shape=jax.ShapeDtypeStruct((n, H), jnp.float32),
    )(pos1c, pos2c, y)

    return out.reshape(batch, seq, hidden), aux[0, 0]
